# baseline (speedup 1.0000x reference)
import functools

import jax
import jax.numpy as jnp
from jax import lax
from jax.experimental import pallas as pl
from jax.experimental.pallas import tpu as pltpu
from jax.experimental.pallas import tpu_sc as plsc

N_NODES = 10000
E_EDGES = 160000
LANE = 16
NC, NS = 2, 16
NW = NC * NS
ROWS_PAD = 10240
ROWS_PER = ROWS_PAD // NW
CH = 1024
NCHUNK = 160
E_PAD = NCHUNK * CH
EROWS = NCHUNK * 24
SROWS = NS * ROWS_PER
SHR_ROWS = max(EROWS, SROWS)
PAD_DST = ROWS_PAD - 1
GB = 96
CHP = 1056


def _seg_reduce_body(is_max, x_hbm, edges_hbm, out_hbm,
                     shr, acc, eb0, eb1, srcc, ldstc, wc, st0, st1,
                     esem0, esem1, gsem0, gsem1):
    cid = lax.axis_index("c")
    sid = lax.axis_index("s")
    lo = (cid * NS + sid) * ROWS_PER

    init_val = -jnp.inf if is_max else 0.0
    iota = lax.iota(jnp.int32, LANE)

    erows = EROWS // NS
    pltpu.sync_copy(edges_hbm.at[pl.ds(sid * erows, erows), :],
                    shr.at[pl.ds(sid * erows, erows), :])

    def init_body(i, _):
        r = i // 8
        j = i % 8
        acc[r, pl.ds(j * LANE, LANE)] = jnp.full((LANE,), init_val,
                                                 jnp.float32)
        return 0

    lax.fori_loop(0, (ROWS_PER + 1) * 8, init_body, 0)

    def zero_idx(i, _):
        srcc[pl.ds(i * LANE, LANE)] = jnp.zeros((LANE,), jnp.int32)
        ldstc[pl.ds(i * LANE, LANE)] = jnp.zeros((LANE,), jnp.int32)
        wc[pl.ds(i * LANE, LANE)] = jnp.zeros((LANE,), jnp.float32)
        return 0

    lax.fori_loop(0, CHP // LANE, zero_idx, 0)

    plsc.subcore_barrier()

    def issue_chunk(ebuf, esem, ci):
        pltpu.async_copy(shr.at[pl.ds(ci * 24, 24), :], ebuf, esem)

    def wait_chunk(ebuf, esem, ci):
        pltpu.make_async_copy(shr.at[pl.ds(ci * 24, 24), :],
                              ebuf, esem).wait()

    def issue_gather(stage, gsem, g0):
        pltpu.async_copy(x_hbm.at[srcc.at[pl.ds(g0, GB)]], stage, gsem)

    def wait_gather(stage, gsem, g0):
        pltpu.make_async_copy(x_hbm.at[srcc.at[pl.ds(g0, GB)]],
                              stage, gsem).wait()

    def process_batch(stage, g0, m_pad):
        n_grp = jnp.minimum(m_pad - g0, GB) // LANE

        def grp_body(gi, _):
            eb = gi * LANE
            ld16 = ldstc[pl.ds(g0 + eb, LANE)]
            w16 = wc[pl.ds(g0 + eb, LANE)]
            for lane in range(LANE):
                rowv = jnp.broadcast_to(ld16[lane], (LANE,))
                wbv = jnp.broadcast_to(w16[lane], (LANE,))
                for j in range(8):
                    colv = (j * LANE) + iota
                    g = stage[eb + lane, pl.ds(j * LANE, LANE)]
                    a = plsc.load_gather(acc, [rowv, colv])
                    msg = g * wbv
                    upd = jnp.maximum(a, msg) if is_max else a + msg
                    plsc.store_scatter(acc, [rowv, colv], upd)
            return 0

        lax.fori_loop(0, n_grp, grp_body, 0)

    def process_chunk(ebuf, _ci):
        def scan_body(i, mv):
            r = i // 8
            c = (i % 8) * LANE
            sv = plsc.bitcast(ebuf[r, pl.ds(c, LANE)], jnp.int32)
            dv = plsc.bitcast(ebuf[r + 8, pl.ds(c, LANE)], jnp.int32)
            wv = ebuf[r + 16, pl.ds(c, LANE)]
            ldv = dv - lo
            msk = (ldv >= 0) & (ldv < ROWS_PER)
            inc = lax.cumsum(msk.astype(jnp.int32))
            pos = jnp.maximum(mv + inc - 1, 0)
            plsc.store_scatter(srcc, [pos], sv, mask=msk)
            plsc.store_scatter(ldstc, [pos], ldv, mask=msk)
            plsc.store_scatter(wc, [pos], wv, mask=msk)
            return mv + plsc.all_reduce_population_count(msk)

        mv = lax.fori_loop(0, CH // LANE, scan_body,
                           jnp.zeros((LANE,), jnp.int32))
        plsc.store_scatter(srcc, [mv + iota], jnp.zeros((LANE,), jnp.int32))
        plsc.store_scatter(ldstc, [mv + iota],
                           jnp.full((LANE,), ROWS_PER, jnp.int32))
        plsc.store_scatter(wc, [mv + iota], jnp.zeros((LANE,), jnp.float32))
        m = mv[0]
        m_pad = ((m + (LANE - 1)) // LANE) * LANE
        nb = (m_pad + (GB - 1)) // GB

        @pl.when(nb > 0)
        def _():
            issue_gather(st0, gsem0, 0)

            def batch_pair(i, _):
                b0 = 2 * i
                g0 = b0 * GB
                wait_gather(st0, gsem0, g0)

                @pl.when(b0 + 1 < nb)
                def _():
                    issue_gather(st1, gsem1, g0 + GB)

                process_batch(st0, g0, m_pad)

                @pl.when(b0 + 1 < nb)
                def _():
                    wait_gather(st1, gsem1, g0 + GB)

                    @pl.when(b0 + 2 < nb)
                    def _():
                        issue_gather(st0, gsem0, g0 + 2 * GB)

                    process_batch(st1, g0 + GB, m_pad)

                return 0

            lax.fori_loop(0, (nb + 1) // 2, batch_pair, 0)

    issue_chunk(eb0, esem0, 0)

    def chunk_pair(i, _):
        c0 = 2 * i
        wait_chunk(eb0, esem0, c0)
        issue_chunk(eb1, esem1, c0 + 1)
        process_chunk(eb0, c0)
        wait_chunk(eb1, esem1, c0 + 1)

        @pl.when(c0 + 2 < NCHUNK)
        def _():
            issue_chunk(eb0, esem0, c0 + 2)

        process_chunk(eb1, c0 + 1)
        return 0

    lax.fori_loop(0, NCHUNK // 2, chunk_pair, 0)

    if is_max:
        def fix_body(i, _):
            r = i // 8
            j = i % 8
            v = acc[r, pl.ds(j * LANE, LANE)]
            acc[r, pl.ds(j * LANE, LANE)] = jnp.where(
                v == -jnp.inf, jnp.zeros((LANE,), jnp.float32), v)
            return 0

        lax.fori_loop(0, ROWS_PER * 8, fix_body, 0)

    plsc.subcore_barrier()
    pltpu.sync_copy(acc.at[pl.ds(0, ROWS_PER), :],
                    shr.at[pl.ds(sid * ROWS_PER, ROWS_PER), :])
    pltpu.sync_copy(shr.at[pl.ds(sid * ROWS_PER, ROWS_PER), :],
                    out_hbm.at[pl.ds(lo, ROWS_PER), :])


def _seg_reduce(x_pad, edges_packed, *, is_max):
    mesh = plsc.VectorSubcoreMesh(core_axis_name="c", subcore_axis_name="s")
    body = functools.partial(_seg_reduce_body, is_max)
    fn = pl.kernel(
        body,
        out_type=jax.ShapeDtypeStruct((ROWS_PAD, 128), jnp.float32),
        mesh=mesh,
        scratch_types=[
            pltpu.VMEM_SHARED((SHR_ROWS, 128), jnp.float32),
            pltpu.VMEM((ROWS_PER + 1, 128), jnp.float32),
            pltpu.VMEM((24, 128), jnp.float32),
            pltpu.VMEM((24, 128), jnp.float32),
            pltpu.VMEM((CHP,), jnp.int32),
            pltpu.VMEM((CHP,), jnp.int32),
            pltpu.VMEM((CHP,), jnp.float32),
            pltpu.VMEM((GB, 128), jnp.float32),
            pltpu.VMEM((GB, 128), jnp.float32),
            pltpu.SemaphoreType.DMA,
            pltpu.SemaphoreType.DMA,
            pltpu.SemaphoreType.DMA,
            pltpu.SemaphoreType.DMA,
        ],
        name="seg_max" if is_max else "seg_sum",
        compiler_params=pltpu.CompilerParams(needs_layout_passes=False),
    )
    return fn(x_pad, edges_packed)


def _pack_edges(src, dst, w):
    pad = E_PAD - E_EDGES
    src_p = jnp.concatenate([src, jnp.zeros((pad,), jnp.int32)])
    dst_p = jnp.concatenate([dst, jnp.full((pad,), PAD_DST, jnp.int32)])
    w_p = jnp.concatenate([lax.bitcast_convert_type(w, jnp.int32),
                           jnp.zeros((pad,), jnp.int32)])
    s3 = src_p.reshape(NCHUNK, 8, 128)
    d3 = dst_p.reshape(NCHUNK, 8, 128)
    w3 = w_p.reshape(NCHUNK, 8, 128)
    packed = jnp.concatenate([s3, d3, w3], axis=1).reshape(EROWS, 128)
    return lax.bitcast_convert_type(packed, jnp.float32)


def _pad_rows(x):
    return jnp.concatenate(
        [x, jnp.zeros((ROWS_PAD - x.shape[0], x.shape[1]), x.dtype)], axis=0)


_BR = 1000


def _mm_body(a_ref, w_ref, o_ref):
    o_ref[...] = jnp.dot(a_ref[...], w_ref[...],
                         preferred_element_type=jnp.float32)


def _mm(a, w):
    m, k = a.shape
    n = w.shape[1]
    assert m % _BR == 0
    return pl.pallas_call(
        _mm_body,
        grid=(m // _BR,),
        in_specs=[pl.BlockSpec((_BR, k), lambda i: (i, 0)),
                  pl.BlockSpec((k, n), lambda i: (0, 0))],
        out_specs=pl.BlockSpec((_BR, n), lambda i: (i, 0)),
        out_shape=jax.ShapeDtypeStruct((m, n), jnp.float32),
    )(a, w)


def _fused_body(c_ref, a_ref, w_ref, b_ref, o_ref):
    x = c_ref[...] + jnp.dot(a_ref[...], w_ref[...],
                             preferred_element_type=jnp.float32) + b_ref[...]
    o_ref[...] = jnp.where(x >= 0, x, 0.01 * x)


def _fused(c, a, w, b):
    m, k = a.shape
    n = w.shape[1]
    assert m % _BR == 0 and c.shape == (m, n)
    return pl.pallas_call(
        _fused_body,
        grid=(m // _BR,),
        in_specs=[pl.BlockSpec((_BR, n), lambda i: (i, 0)),
                  pl.BlockSpec((_BR, k), lambda i: (i, 0)),
                  pl.BlockSpec((k, n), lambda i: (0, 0)),
                  pl.BlockSpec((1, n), lambda i: (0, 0))],
        out_specs=pl.BlockSpec((_BR, n), lambda i: (i, 0)),
        out_shape=jax.ShapeDtypeStruct((m, n), jnp.float32),
    )(c, a, w, b.reshape(1, n))


def kernel(x_metric, x_alert, edge_index_corr, edge_index_cause,
           edge_weight_corr, edge_weight_cause,
           W_rel_corr_0, b_rel_corr_0, W_root_corr_0,
           W_rel_cause_0, b_rel_cause_0, W_root_cause_0,
           W_rel_corr_1, b_rel_corr_1, W_root_corr_1,
           W_rel_cause_1, b_rel_cause_1, W_root_cause_1):
    xm = x_metric
    xa = x_alert
    ec = _pack_edges(edge_index_corr[0].astype(jnp.int32),
                     edge_index_corr[1].astype(jnp.int32), edge_weight_corr)
    ea = _pack_edges(edge_index_cause[0].astype(jnp.int32),
                     edge_index_cause[1].astype(jnp.int32), edge_weight_cause)

    p = _mm(xm, W_rel_corr_0)
    r_a = _mm(xa, W_root_cause_0)

    s = _seg_reduce(_pad_rows(p), ec, is_max=False)[:N_NODES]
    m0a = _seg_reduce(_pad_rows(xm[:, :128]), ea, is_max=True)
    m0b = _seg_reduce(_pad_rows(xm[:, 128:]), ea, is_max=True)
    m0 = jnp.concatenate([m0a[:N_NODES], m0b[:N_NODES]], axis=1)

    xm1 = _fused(s, xm, W_root_corr_0, b_rel_corr_0)
    xa1 = _fused(r_a, m0, W_rel_cause_0, b_rel_cause_0)

    m1 = _seg_reduce(_pad_rows(xm1), ea, is_max=True)[:N_NODES]
    t = _mm(m1, W_rel_cause_1)
    xa2 = _fused(t, xa1, W_root_cause_1, b_rel_cause_1)
    return xa2


# speedup vs baseline: 6.8424x; 6.8424x over previous
import functools

import jax
import jax.numpy as jnp
from jax import lax
from jax.experimental import pallas as pl
from jax.experimental.pallas import tpu as pltpu
from jax.experimental.pallas import tpu_sc as plsc

N_NODES = 10000
E_EDGES = 160000
LANE = 16
NC, NS = 2, 16
NW = NC * NS
ROWS_PAD = 10240
ROWS_PER = ROWS_PAD // NW
CH = 5120
CROWS = (CH // 128) * 2
NCHUNK = 32
E_PAD = NCHUNK * CH
EROWS = NCHUNK * CROWS
PAD_DST = ROWS_PAD - 1
GB = 192
CHP = 5376


def _seg_reduce_body(is_max, x_hbm, edges_hbm, out_hbm,
                     acc, eb0, eb1, srcc, ldstc, wc, stage,
                     esem0, esem1, gsem):
    cid = lax.axis_index("c")
    sid = lax.axis_index("s")
    lo = (cid * NS + sid) * ROWS_PER

    init_val = -jnp.inf if is_max else 0.0
    iota = lax.iota(jnp.int32, LANE)

    def init_body(i, _):
        r = i // 8
        j = i % 8
        acc[r, pl.ds(j * LANE, LANE)] = jnp.full((LANE,), init_val,
                                                 jnp.float32)
        return 0

    lax.fori_loop(0, (ROWS_PER + 1) * 8, init_body, 0)

    def zero_idx(i, _):
        srcc[pl.ds(i * LANE, LANE)] = jnp.zeros((LANE,), jnp.int32)
        ldstc[pl.ds(i * LANE, LANE)] = jnp.zeros((LANE,), jnp.int32)
        wc[pl.ds(i * LANE, LANE)] = jnp.zeros((LANE,), jnp.float32)
        return 0

    lax.fori_loop(0, CHP // LANE, zero_idx, 0)

    def issue_chunk(ebuf, esem, ci):
        pltpu.async_copy(edges_hbm.at[pl.ds(ci * CROWS, CROWS), :],
                         ebuf, esem)

    def wait_chunk(ebuf, esem, ci):
        pltpu.make_async_copy(edges_hbm.at[pl.ds(ci * CROWS, CROWS), :],
                              ebuf, esem).wait()

    def process_batch(g0, m_pad):
        n_grp = jnp.minimum(m_pad - g0, GB) // LANE

        def grp_body(gi, _):
            eb = gi * LANE
            ld16 = ldstc[pl.ds(g0 + eb, LANE)]
            w16 = wc[pl.ds(g0 + eb, LANE)]
            for lane in range(LANE):
                rowv = jnp.broadcast_to(ld16[lane], (LANE,))
                wbv = jnp.broadcast_to(w16[lane], (LANE,))
                for j in range(8):
                    colv = (j * LANE) + iota
                    g = stage[eb + lane, pl.ds(j * LANE, LANE)]
                    a = plsc.load_gather(acc, [rowv, colv])
                    msg = g * wbv
                    upd = jnp.maximum(a, msg) if is_max else a + msg
                    plsc.store_scatter(acc, [rowv, colv], upd)
            return 0

        lax.fori_loop(0, n_grp, grp_body, 0)

    def process_chunk(ebuf, _ci):
        def scan_body(i, mv):
            r = i // 8
            c = (i % 8) * LANE
            pv = ebuf[r, pl.ds(c, LANE)]
            wv = plsc.bitcast(ebuf[r + CROWS // 2, pl.ds(c, LANE)],
                              jnp.float32)
            sv = pv & 0x3FFF
            dv = lax.shift_right_logical(pv, 14)
            ldv = dv - lo
            msk = (ldv >= 0) & (ldv < ROWS_PER)
            inc = lax.cumsum(msk.astype(jnp.int32))
            pos = jnp.maximum(mv + inc - 1, 0)
            plsc.store_scatter(srcc, [pos], sv, mask=msk)
            plsc.store_scatter(ldstc, [pos], ldv, mask=msk)
            plsc.store_scatter(wc, [pos], wv, mask=msk)
            return mv + plsc.all_reduce_population_count(msk)

        mv = lax.fori_loop(0, CH // LANE, scan_body,
                           jnp.zeros((LANE,), jnp.int32))
        plsc.store_scatter(srcc, [mv + iota], jnp.zeros((LANE,), jnp.int32))
        plsc.store_scatter(ldstc, [mv + iota],
                           jnp.full((LANE,), ROWS_PER, jnp.int32))
        plsc.store_scatter(wc, [mv + iota], jnp.zeros((LANE,), jnp.float32))
        m = mv[0]
        m_pad = ((m + (LANE - 1)) // LANE) * LANE
        nb = (m_pad + (GB - 1)) // GB

        def batch_body(b, _):
            g0 = b * GB
            pltpu.async_copy(x_hbm.at[srcc.at[pl.ds(g0, GB)]], stage,
                             gsem)
            pltpu.make_async_copy(x_hbm.at[srcc.at[pl.ds(g0, GB)]],
                                  stage, gsem).wait()
            process_batch(g0, m_pad)
            return 0

        lax.fori_loop(0, nb, batch_body, 0)

    issue_chunk(eb0, esem0, 0)

    def chunk_pair(i, _):
        c0 = 2 * i
        wait_chunk(eb0, esem0, c0)
        issue_chunk(eb1, esem1, c0 + 1)
        process_chunk(eb0, c0)
        wait_chunk(eb1, esem1, c0 + 1)

        @pl.when(c0 + 2 < NCHUNK)
        def _():
            issue_chunk(eb0, esem0, c0 + 2)

        process_chunk(eb1, c0 + 1)
        return 0

    lax.fori_loop(0, NCHUNK // 2, chunk_pair, 0)

    if is_max:
        def fix_body(i, _):
            r = i // 8
            j = i % 8
            v = acc[r, pl.ds(j * LANE, LANE)]
            acc[r, pl.ds(j * LANE, LANE)] = jnp.where(
                v == -jnp.inf, jnp.zeros((LANE,), jnp.float32), v)
            return 0

        lax.fori_loop(0, ROWS_PER * 8, fix_body, 0)

    pltpu.sync_copy(acc.at[pl.ds(0, ROWS_PER), :],
                    out_hbm.at[pl.ds(lo, ROWS_PER), :])


def _seg_reduce(x_pad, edges_packed, *, is_max):
    mesh = plsc.VectorSubcoreMesh(core_axis_name="c", subcore_axis_name="s")
    body = functools.partial(_seg_reduce_body, is_max)
    fn = pl.kernel(
        body,
        out_type=jax.ShapeDtypeStruct((ROWS_PAD, 128), jnp.float32),
        mesh=mesh,
        scratch_types=[
            pltpu.VMEM((ROWS_PER + 1, 128), jnp.float32),
            pltpu.VMEM((CROWS, 128), jnp.int32),
            pltpu.VMEM((CROWS, 128), jnp.int32),
            pltpu.VMEM((CHP,), jnp.int32),
            pltpu.VMEM((CHP,), jnp.int32),
            pltpu.VMEM((CHP,), jnp.float32),
            pltpu.VMEM((GB, 128), jnp.float32),
            pltpu.SemaphoreType.DMA,
            pltpu.SemaphoreType.DMA,
            pltpu.SemaphoreType.DMA,
        ],
        name="seg_max" if is_max else "seg_sum",
        compiler_params=pltpu.CompilerParams(needs_layout_passes=False),
    )
    return fn(x_pad, edges_packed)


def _pack_edges(src, dst, w):
    pad = E_PAD - E_EDGES
    src_p = jnp.concatenate([src, jnp.zeros((pad,), jnp.int32)])
    dst_p = jnp.concatenate([dst, jnp.full((pad,), PAD_DST, jnp.int32)])
    pd = src_p | (dst_p << 14)
    w_p = jnp.concatenate([lax.bitcast_convert_type(w, jnp.int32),
                           jnp.zeros((pad,), jnp.int32)])
    p3 = pd.reshape(NCHUNK, CROWS // 2, 128)
    w3 = w_p.reshape(NCHUNK, CROWS // 2, 128)
    return jnp.concatenate([p3, w3], axis=1).reshape(EROWS, 128)


def _pad_rows(x):
    return jnp.concatenate(
        [x, jnp.zeros((ROWS_PAD - x.shape[0], x.shape[1]), x.dtype)], axis=0)


_BR = 1000


def _mm_body(a_ref, w_ref, o_ref):
    o_ref[...] = jnp.dot(a_ref[...], w_ref[...],
                         preferred_element_type=jnp.float32)


def _mm(a, w):
    m, k = a.shape
    n = w.shape[1]
    assert m % _BR == 0
    return pl.pallas_call(
        _mm_body,
        grid=(m // _BR,),
        in_specs=[pl.BlockSpec((_BR, k), lambda i: (i, 0)),
                  pl.BlockSpec((k, n), lambda i: (0, 0))],
        out_specs=pl.BlockSpec((_BR, n), lambda i: (i, 0)),
        out_shape=jax.ShapeDtypeStruct((m, n), jnp.float32),
    )(a, w)


def _fused_body(c_ref, a_ref, w_ref, b_ref, o_ref):
    x = c_ref[...] + jnp.dot(a_ref[...], w_ref[...],
                             preferred_element_type=jnp.float32) + b_ref[...]
    o_ref[...] = jnp.where(x >= 0, x, 0.01 * x)


def _fused(c, a, w, b):
    m, k = a.shape
    n = w.shape[1]
    assert m % _BR == 0 and c.shape == (m, n)
    return pl.pallas_call(
        _fused_body,
        grid=(m // _BR,),
        in_specs=[pl.BlockSpec((_BR, n), lambda i: (i, 0)),
                  pl.BlockSpec((_BR, k), lambda i: (i, 0)),
                  pl.BlockSpec((k, n), lambda i: (0, 0)),
                  pl.BlockSpec((1, n), lambda i: (0, 0))],
        out_specs=pl.BlockSpec((_BR, n), lambda i: (i, 0)),
        out_shape=jax.ShapeDtypeStruct((m, n), jnp.float32),
    )(c, a, w, b.reshape(1, n))


def kernel(x_metric, x_alert, edge_index_corr, edge_index_cause,
           edge_weight_corr, edge_weight_cause,
           W_rel_corr_0, b_rel_corr_0, W_root_corr_0,
           W_rel_cause_0, b_rel_cause_0, W_root_cause_0,
           W_rel_corr_1, b_rel_corr_1, W_root_corr_1,
           W_rel_cause_1, b_rel_cause_1, W_root_cause_1):
    xm = x_metric
    xa = x_alert
    ec = _pack_edges(edge_index_corr[0].astype(jnp.int32),
                     edge_index_corr[1].astype(jnp.int32), edge_weight_corr)
    ea = _pack_edges(edge_index_cause[0].astype(jnp.int32),
                     edge_index_cause[1].astype(jnp.int32), edge_weight_cause)

    p = _mm(xm, W_rel_corr_0)
    r_a = _mm(xa, W_root_cause_0)

    s = _seg_reduce(_pad_rows(p), ec, is_max=False)[:N_NODES]
    m0a = _seg_reduce(_pad_rows(xm[:, :128]), ea, is_max=True)
    m0b = _seg_reduce(_pad_rows(xm[:, 128:]), ea, is_max=True)
    m0 = jnp.concatenate([m0a[:N_NODES], m0b[:N_NODES]], axis=1)

    xm1 = _fused(s, xm, W_root_corr_0, b_rel_corr_0)
    xa1 = _fused(r_a, m0, W_rel_cause_0, b_rel_cause_0)

    m1 = _seg_reduce(_pad_rows(xm1), ea, is_max=True)[:N_NODES]
    t = _mm(m1, W_rel_cause_1)
    xa2 = _fused(t, xa1, W_root_cause_1, b_rel_cause_1)
    return xa2


# speedup vs baseline: 7.4486x; 1.0886x over previous
import functools

import jax
import jax.numpy as jnp
from jax import lax
from jax.experimental import pallas as pl
from jax.experimental.pallas import tpu as pltpu
from jax.experimental.pallas import tpu_sc as plsc

N_NODES = 10000
E_EDGES = 160000
LANE = 16
NC, NS = 2, 16
NW = NC * NS
ROWS_PAD = 10240
ROWS_PER = ROWS_PAD // NW
CH = 5120
CROWS = (CH // 128) * 2
NCHUNK = 32
E_PAD = NCHUNK * CH
EROWS = NCHUNK * CROWS
PAD_DST = ROWS_PAD - 1
GB = 192
CHP = 5376


def _seg_reduce_body(is_max, x_hbm, edges_hbm, out_hbm,
                     acc, eb0, eb1, srcc, ldstc, wc, st0, st1,
                     esem0, esem1, gsem0, gsem1):
    cid = lax.axis_index("c")
    sid = lax.axis_index("s")
    lo = (cid * NS + sid) * ROWS_PER

    init_val = -jnp.inf if is_max else 0.0
    iota = lax.iota(jnp.int32, LANE)

    def init_body(i, _):
        r = i // 8
        j = i % 8
        acc[r, pl.ds(j * LANE, LANE)] = jnp.full((LANE,), init_val,
                                                 jnp.float32)
        return 0

    lax.fori_loop(0, (ROWS_PER + 1) * 8, init_body, 0)

    def zero_idx(i, _):
        srcc[pl.ds(i * LANE, LANE)] = jnp.zeros((LANE,), jnp.int32)
        ldstc[pl.ds(i * LANE, LANE)] = jnp.zeros((LANE,), jnp.int32)
        wc[pl.ds(i * LANE, LANE)] = jnp.zeros((LANE,), jnp.float32)
        return 0

    lax.fori_loop(0, CHP // LANE, zero_idx, 0)

    def issue_chunk(ebuf, esem, ci):
        pltpu.async_copy(edges_hbm.at[pl.ds(ci * CROWS, CROWS), :],
                         ebuf, esem)

    def wait_chunk(ebuf, esem, ci):
        pltpu.make_async_copy(edges_hbm.at[pl.ds(ci * CROWS, CROWS), :],
                              ebuf, esem).wait()

    def process_batch(stage, g0, m_pad):
        n_grp = jnp.minimum(m_pad - g0, GB) // LANE

        def grp_body(gi, _):
            eb = gi * LANE
            ld16 = ldstc[pl.ds(g0 + eb, LANE)]
            w16 = wc[pl.ds(g0 + eb, LANE)]
            for lane in range(LANE):
                rowv = jnp.broadcast_to(ld16[lane], (LANE,))
                wbv = jnp.broadcast_to(w16[lane], (LANE,))
                for j in range(8):
                    colv = (j * LANE) + iota
                    g = stage[eb + lane, pl.ds(j * LANE, LANE)]
                    a = plsc.load_gather(acc, [rowv, colv])
                    msg = g * wbv
                    upd = jnp.maximum(a, msg) if is_max else a + msg
                    plsc.store_scatter(acc, [rowv, colv], upd)
            return 0

        lax.fori_loop(0, n_grp, grp_body, 0)

    def process_chunk(ebuf, _ci):
        def scan_body(i, mv):
            r = i // 8
            c = (i % 8) * LANE
            pv = ebuf[r, pl.ds(c, LANE)]
            wv = plsc.bitcast(ebuf[r + CROWS // 2, pl.ds(c, LANE)],
                              jnp.float32)
            sv = pv & 0x3FFF
            dv = lax.shift_right_logical(pv, 14)
            ldv = dv - lo
            msk = (ldv >= 0) & (ldv < ROWS_PER)
            inc = lax.cumsum(msk.astype(jnp.int32))
            pos = jnp.maximum(mv + inc - 1, 0)
            plsc.store_scatter(srcc, [pos], sv, mask=msk)
            plsc.store_scatter(ldstc, [pos], ldv, mask=msk)
            plsc.store_scatter(wc, [pos], wv, mask=msk)
            return mv + plsc.all_reduce_population_count(msk)

        mv = lax.fori_loop(0, CH // LANE, scan_body,
                           jnp.zeros((LANE,), jnp.int32))
        plsc.store_scatter(srcc, [mv + iota], jnp.zeros((LANE,), jnp.int32))
        plsc.store_scatter(ldstc, [mv + iota],
                           jnp.full((LANE,), ROWS_PER, jnp.int32))
        plsc.store_scatter(wc, [mv + iota], jnp.zeros((LANE,), jnp.float32))
        m = mv[0]
        m_pad = ((m + (LANE - 1)) // LANE) * LANE
        nb = (m_pad + (GB - 1)) // GB

        def issue_gather(stage, gsem, g0):
            pltpu.async_copy(x_hbm.at[srcc.at[pl.ds(g0, GB)]], stage, gsem)

        def wait_gather(stage, gsem, g0):
            pltpu.make_async_copy(x_hbm.at[srcc.at[pl.ds(g0, GB)]],
                                  stage, gsem).wait()

        @pl.when(nb > 0)
        def _():
            issue_gather(st0, gsem0, 0)

            def batch_pair(i, _):
                b0 = 2 * i
                g0 = b0 * GB

                @pl.when(b0 + 1 < nb)
                def _():
                    issue_gather(st1, gsem1, g0 + GB)

                wait_gather(st0, gsem0, g0)
                process_batch(st0, g0, m_pad)

                @pl.when(b0 + 2 < nb)
                def _():
                    issue_gather(st0, gsem0, g0 + 2 * GB)

                @pl.when(b0 + 1 < nb)
                def _():
                    wait_gather(st1, gsem1, g0 + GB)
                    process_batch(st1, g0 + GB, m_pad)

                return 0

            lax.fori_loop(0, (nb + 1) // 2, batch_pair, 0)

    issue_chunk(eb0, esem0, 0)
    issue_chunk(eb1, esem1, 1)

    def chunk_pair(i, _):
        c0 = 2 * i
        wait_chunk(eb0, esem0, c0)
        process_chunk(eb0, c0)

        @pl.when(c0 + 2 < NCHUNK)
        def _():
            issue_chunk(eb0, esem0, c0 + 2)

        wait_chunk(eb1, esem1, c0 + 1)
        process_chunk(eb1, c0 + 1)

        @pl.when(c0 + 3 < NCHUNK)
        def _():
            issue_chunk(eb1, esem1, c0 + 3)

        return 0

    lax.fori_loop(0, NCHUNK // 2, chunk_pair, 0)

    if is_max:
        def fix_body(i, _):
            r = i // 8
            j = i % 8
            v = acc[r, pl.ds(j * LANE, LANE)]
            acc[r, pl.ds(j * LANE, LANE)] = jnp.where(
                v == -jnp.inf, jnp.zeros((LANE,), jnp.float32), v)
            return 0

        lax.fori_loop(0, ROWS_PER * 8, fix_body, 0)

    pltpu.sync_copy(acc.at[pl.ds(0, ROWS_PER), :],
                    out_hbm.at[pl.ds(lo, ROWS_PER), :])


def _seg_reduce(x_pad, edges_packed, *, is_max):
    mesh = plsc.VectorSubcoreMesh(core_axis_name="c", subcore_axis_name="s")
    body = functools.partial(_seg_reduce_body, is_max)
    fn = pl.kernel(
        body,
        out_type=jax.ShapeDtypeStruct((ROWS_PAD, 128), jnp.float32),
        mesh=mesh,
        scratch_types=[
            pltpu.VMEM((ROWS_PER + 1, 128), jnp.float32),
            pltpu.VMEM((CROWS, 128), jnp.int32),
            pltpu.VMEM((CROWS, 128), jnp.int32),
            pltpu.VMEM((CHP,), jnp.int32),
            pltpu.VMEM((CHP,), jnp.int32),
            pltpu.VMEM((CHP,), jnp.float32),
            pltpu.VMEM((GB, 128), jnp.float32),
            pltpu.VMEM((GB, 128), jnp.float32),
            pltpu.SemaphoreType.DMA,
            pltpu.SemaphoreType.DMA,
            pltpu.SemaphoreType.DMA,
            pltpu.SemaphoreType.DMA,
        ],
        name="seg_max" if is_max else "seg_sum",
        compiler_params=pltpu.CompilerParams(needs_layout_passes=False),
    )
    return fn(x_pad, edges_packed)


def _pack_edges(src, dst, w):
    pad = E_PAD - E_EDGES
    src_p = jnp.concatenate([src, jnp.zeros((pad,), jnp.int32)])
    dst_p = jnp.concatenate([dst, jnp.full((pad,), PAD_DST, jnp.int32)])
    pd = src_p | (dst_p << 14)
    w_p = jnp.concatenate([lax.bitcast_convert_type(w, jnp.int32),
                           jnp.zeros((pad,), jnp.int32)])
    p3 = pd.reshape(NCHUNK, CROWS // 2, 128)
    w3 = w_p.reshape(NCHUNK, CROWS // 2, 128)
    return jnp.concatenate([p3, w3], axis=1).reshape(EROWS, 128)


def _pad_rows(x):
    return jnp.concatenate(
        [x, jnp.zeros((ROWS_PAD - x.shape[0], x.shape[1]), x.dtype)], axis=0)


_BR = 1000


def _mm_body(a_ref, w_ref, o_ref):
    o_ref[...] = jnp.dot(a_ref[...], w_ref[...],
                         preferred_element_type=jnp.float32)


def _mm(a, w):
    m, k = a.shape
    n = w.shape[1]
    assert m % _BR == 0
    return pl.pallas_call(
        _mm_body,
        grid=(m // _BR,),
        in_specs=[pl.BlockSpec((_BR, k), lambda i: (i, 0)),
                  pl.BlockSpec((k, n), lambda i: (0, 0))],
        out_specs=pl.BlockSpec((_BR, n), lambda i: (i, 0)),
        out_shape=jax.ShapeDtypeStruct((m, n), jnp.float32),
    )(a, w)


def _fused_body(c_ref, a_ref, w_ref, b_ref, o_ref):
    x = c_ref[...] + jnp.dot(a_ref[...], w_ref[...],
                             preferred_element_type=jnp.float32) + b_ref[...]
    o_ref[...] = jnp.where(x >= 0, x, 0.01 * x)


def _fused(c, a, w, b):
    m, k = a.shape
    n = w.shape[1]
    assert m % _BR == 0 and c.shape == (m, n)
    return pl.pallas_call(
        _fused_body,
        grid=(m // _BR,),
        in_specs=[pl.BlockSpec((_BR, n), lambda i: (i, 0)),
                  pl.BlockSpec((_BR, k), lambda i: (i, 0)),
                  pl.BlockSpec((k, n), lambda i: (0, 0)),
                  pl.BlockSpec((1, n), lambda i: (0, 0))],
        out_specs=pl.BlockSpec((_BR, n), lambda i: (i, 0)),
        out_shape=jax.ShapeDtypeStruct((m, n), jnp.float32),
    )(c, a, w, b.reshape(1, n))


def kernel(x_metric, x_alert, edge_index_corr, edge_index_cause,
           edge_weight_corr, edge_weight_cause,
           W_rel_corr_0, b_rel_corr_0, W_root_corr_0,
           W_rel_cause_0, b_rel_cause_0, W_root_cause_0,
           W_rel_corr_1, b_rel_corr_1, W_root_corr_1,
           W_rel_cause_1, b_rel_cause_1, W_root_cause_1):
    xm = x_metric
    xa = x_alert
    ec = _pack_edges(edge_index_corr[0].astype(jnp.int32),
                     edge_index_corr[1].astype(jnp.int32), edge_weight_corr)
    ea = _pack_edges(edge_index_cause[0].astype(jnp.int32),
                     edge_index_cause[1].astype(jnp.int32), edge_weight_cause)

    p = _mm(xm, W_rel_corr_0)
    r_a = _mm(xa, W_root_cause_0)

    s = _seg_reduce(_pad_rows(p), ec, is_max=False)[:N_NODES]
    m0a = _seg_reduce(_pad_rows(xm[:, :128]), ea, is_max=True)
    m0b = _seg_reduce(_pad_rows(xm[:, 128:]), ea, is_max=True)
    m0 = jnp.concatenate([m0a[:N_NODES], m0b[:N_NODES]], axis=1)

    xm1 = _fused(s, xm, W_root_corr_0, b_rel_corr_0)
    xa1 = _fused(r_a, m0, W_rel_cause_0, b_rel_cause_0)

    m1 = _seg_reduce(_pad_rows(xm1), ea, is_max=True)[:N_NODES]
    t = _mm(m1, W_rel_cause_1)
    xa2 = _fused(t, xa1, W_root_cause_1, b_rel_cause_1)
    return xa2
